# Initial kernel scaffold; baseline (speedup 1.0000x reference)
import jax
import jax.numpy as jnp
from jax.experimental import pallas as pl


def kernel(x, codebook, ema_cluster_hidden, ema_cluster_count, ema_hidden, ema_count):
    raise NotImplementedError("write your pallas kernel here")


# speedup vs baseline: 4.3598x; 4.3598x over previous
import functools
import math

import jax
import jax.numpy as jnp
from jax import lax
from jax.experimental import pallas as pl
from jax.experimental.pallas import tpu as pltpu
from jax.experimental.pallas import tpu_sc as plsc

B, L, K, D = 4, 196, 1024, 256
N = B * L
DECAY = 0.99
EPS = 1e-05

TILE = 112
NTILE = N // TILE

NC, NS = 2, 16
LANES = 16
NPAD = 800
NTOK_CORE = NPAD // NC
KPAD = K + 8


def _argmin_body(x_ref, cbt_ref, cb_ref, idx_ref):
    x = x_ref[...]
    cbt = cbt_ref[...]
    cb = cb_ref[...]
    xc = lax.dot_general(x, cbt, (((1,), (0,)), ((), ())),
                         preferred_element_type=jnp.float32)
    cnorm2 = jnp.sum(cbt * cbt, axis=0)
    scores = cnorm2[None, :] - 2.0 * xc
    iota_k = lax.broadcasted_iota(jnp.int32, (TILE, K), 1)

    rowmin = jnp.min(scores, axis=1, keepdims=True)
    idx1 = jnp.min(jnp.where(scores == rowmin, iota_k, K), axis=1)
    masked = jnp.where(iota_k == idx1[:, None], jnp.inf, scores)
    rowmin2 = jnp.min(masked, axis=1, keepdims=True)
    idx2 = jnp.min(jnp.where(masked == rowmin2, iota_k, K), axis=1)

    def dist(idx):
        enc = (iota_k == idx[:, None]).astype(jnp.float32)
        crow = lax.dot_general(enc, cb, (((1,), (0,)), ((), ())),
                               precision=lax.Precision.HIGHEST,
                               preferred_element_type=jnp.float32)
        return jnp.sqrt(jnp.sum((x - crow) ** 2, axis=1))

    d1, d2 = dist(idx1), dist(idx2)
    take2 = (d2 < d1) | ((d2 == d1) & (idx2 < idx1))
    idx = jnp.where(take2, idx2, idx1)
    idx_ref[...] = idx[:, None]


def _scatter_body(idx_hbm, x_hbm, zdw_hbm, dwp_hbm, idx_l, x_l, dw_l):
    cid = lax.axis_index("c")
    sid = lax.axis_index("s")
    tok0 = cid * NTOK_CORE
    col0 = sid * LANES
    pltpu.sync_copy(idx_hbm.at[pl.ds(tok0, NTOK_CORE)], idx_l)
    pltpu.sync_copy(x_hbm.at[pl.ds(tok0, NTOK_CORE), pl.ds(col0, LANES)], x_l)
    pltpu.sync_copy(zdw_hbm, dw_l)

    def fold(c, carry):
        base = c * LANES
        iv = idx_l[pl.ds(base, LANES)]
        for j in range(LANES):
            k = iv[j]
            dw_l[k, :] = dw_l[k, :] + x_l[base + j, :]
        return carry

    lax.fori_loop(0, NTOK_CORE // LANES, fold, 0)
    pltpu.sync_copy(dw_l, dwp_hbm.at[pl.ds(cid * KPAD, KPAD),
                                     pl.ds(col0, LANES)])


def _update_body(x_ref, idx_ref, dwp_ref, ech_ref, ecc_ref, eh_ref, ec_ref,
                 q_ref, loss_ref):
    x = x_ref[...]
    idx = idx_ref[...]
    iota_k = lax.broadcasted_iota(jnp.int32, (N, K), 1)
    enc = (iota_k == idx).astype(jnp.float32)

    counts = jnp.sum(enc, axis=0)[None, :]
    cc_count_upd = ecc_ref[0, 0] + 1.0
    cc_hidden_upd = ech_ref[...] * DECAY + (1.0 - DECAY) * counts
    cc_avg = cc_hidden_upd / (1.0 - jnp.exp(cc_count_upd * math.log(DECAY)))

    dw = dwp_ref[0:K, :] + dwp_ref[KPAD:KPAD + K, :]
    e_count_upd = ec_ref[0, 0] + 1.0
    e_hidden_upd = eh_ref[...] * DECAY + (1.0 - DECAY) * dw
    e_avg = e_hidden_upd / (1.0 - jnp.exp(e_count_upd * math.log(DECAY)))

    n_tot = jnp.sum(cc_avg)
    counts_n = (cc_avg + EPS) / (n_tot + K * EPS) * n_tot

    q = lax.dot_general(enc / counts_n, e_avg, (((1,), (0,)), ((), ())),
                        preferred_element_type=jnp.float32)
    q = x + (q - x)
    q_ref[...] = q
    loss_ref[0, 0] = jnp.mean(0.5 * (x - q) ** 2)


@functools.cache
def _sc_scatter():
    return pl.kernel(
        _scatter_body,
        mesh=plsc.VectorSubcoreMesh(core_axis_name="c", subcore_axis_name="s"),
        out_type=jax.ShapeDtypeStruct((NC * KPAD, D), jnp.float32),
        scratch_types=[
            pltpu.VMEM((NTOK_CORE,), jnp.int32),
            pltpu.VMEM((NTOK_CORE, LANES), jnp.float32),
            pltpu.VMEM((KPAD, LANES), jnp.float32),
        ],
        compiler_params=pltpu.CompilerParams(use_tc_tiling_on_sc=False),
    )


@jax.jit
def kernel(x, codebook, ema_cluster_hidden, ema_cluster_count, ema_hidden,
           ema_count):
    x2 = x.reshape(N, D)
    cb_t = codebook.T

    idx_col = pl.pallas_call(
        _argmin_body,
        grid=(NTILE,),
        in_specs=[
            pl.BlockSpec((TILE, D), lambda i: (i, 0)),
            pl.BlockSpec((D, K), lambda i: (0, 0)),
            pl.BlockSpec((K, D), lambda i: (0, 0)),
        ],
        out_specs=pl.BlockSpec((TILE, 1), lambda i: (i, 0)),
        out_shape=jax.ShapeDtypeStruct((N, 1), jnp.int32),
    )(x2, cb_t, codebook)

    idx_flat = idx_col.reshape(N)
    idx_pad = jnp.concatenate([idx_flat, jnp.full((NPAD - N,), K, jnp.int32)])
    x_pad = jnp.concatenate([x2, jnp.zeros((NPAD - N, D), jnp.float32)])
    zdw = jnp.zeros((KPAD, LANES), jnp.float32)

    dw_parts = _sc_scatter()(idx_pad, x_pad, zdw)

    q, loss = pl.pallas_call(
        _update_body,
        in_specs=[
            pl.BlockSpec(memory_space=pltpu.VMEM),
            pl.BlockSpec(memory_space=pltpu.VMEM),
            pl.BlockSpec(memory_space=pltpu.VMEM),
            pl.BlockSpec(memory_space=pltpu.VMEM),
            pl.BlockSpec(memory_space=pltpu.SMEM),
            pl.BlockSpec(memory_space=pltpu.VMEM),
            pl.BlockSpec(memory_space=pltpu.SMEM),
        ],
        out_specs=(
            pl.BlockSpec(memory_space=pltpu.VMEM),
            pl.BlockSpec(memory_space=pltpu.SMEM),
        ),
        out_shape=(
            jax.ShapeDtypeStruct((N, D), jnp.float32),
            jax.ShapeDtypeStruct((1, 1), jnp.float32),
        ),
    )(x2, idx_col, dw_parts, ema_cluster_hidden.reshape(1, K),
      ema_cluster_count.reshape(1, 1), ema_hidden, ema_count.reshape(1, 1))

    return (q.reshape(B, L, D), loss.reshape(()), idx_flat.reshape(B, L))
